# Initial kernel scaffold; baseline (speedup 1.0000x reference)
import jax
import jax.numpy as jnp
from jax.experimental import pallas as pl


def kernel(x, edge_index, batch, W1, b1, W2, b2, gamma1, beta1, gamma2, beta2, Wp0, bp0, Wp1, bp1, Wp2, bp2):
    raise NotImplementedError("write your pallas kernel here")


# speedup vs baseline: 9.9278x; 9.9278x over previous
import functools

import jax
import jax.numpy as jnp
from jax import lax
from jax.experimental import pallas as pl
from jax.experimental.pallas import tpu as pltpu
from jax.experimental.pallas import tpu_sc as plsc

N = 10000
D = 128
G = 64

NC = 2
NS = 16
NW = NC * NS
LANES = 16
CH = 128
NPAD = 10240
RPT_ACC = NPAD // NS
RPT_OUT = N // NS

_f32 = jnp.float32
_i32 = jnp.int32


def _sc_mesh():
    return plsc.VectorSubcoreMesh(core_axis_name="c", subcore_axis_name="s")


@functools.lru_cache(maxsize=None)
def _deg_sc(K):
    @functools.partial(
        pl.kernel,
        out_type=jax.ShapeDtypeStruct((NC, NPAD), _f32),
        mesh=_sc_mesh(),
        scratch_types=[
            pltpu.VMEM((K, CH), _i32),
            pltpu.VMEM((CH,), _f32),
            pltpu.VMEM((RPT_ACC,), _f32),
            pltpu.VMEM_SHARED((NPAD,), _f32),
        ],
    )
    def deg_kernel(dst_hbm, out_hbm, dstv, ones_v, zero_v, acc):
        c = lax.axis_index("c")
        s = lax.axis_index("s")
        widx = c * NS + s
        pltpu.sync_copy(dst_hbm.at[widx], dstv)
        for j in range(CH // LANES):
            ones_v[pl.ds(j * LANES, LANES)] = jnp.ones((LANES,), _f32)
        for j in range(RPT_ACC // LANES):
            zero_v[pl.ds(j * LANES, LANES)] = jnp.zeros((LANES,), _f32)
        pltpu.sync_copy(zero_v, acc.at[pl.ds(s * RPT_ACC, RPT_ACC)])
        plsc.subcore_barrier()

        def step(k, carry):
            pltpu.sync_copy(ones_v, acc.at[dstv.at[k]], add=True)
            return carry

        lax.fori_loop(0, K, step, 0)
        plsc.subcore_barrier()
        pltpu.sync_copy(acc.at[pl.ds(s * RPT_ACC, RPT_ACC)],
                        out_hbm.at[c, pl.ds(s * RPT_ACC, RPT_ACC)])

    return deg_kernel


@functools.lru_cache(maxsize=None)
def _spmm_sc(K):
    @functools.partial(
        pl.kernel,
        out_type=jax.ShapeDtypeStruct((NC, NPAD, D), _f32),
        mesh=_sc_mesh(),
        scratch_types=[
            pltpu.VMEM((K, CH), _i32),
            pltpu.VMEM((K, CH), _i32),
            pltpu.VMEM((CH, D), _f32),
            pltpu.VMEM_SHARED((NPAD, D), _f32),
            pltpu.SemaphoreType.DMA,
        ],
    )
    def spmm_kernel(hw_hbm, src_hbm, dst_hbm, out_hbm,
                    srcv, dstv, rows, acc, gsem):
        c = lax.axis_index("c")
        s = lax.axis_index("s")
        widx = c * NS + s
        pltpu.sync_copy(src_hbm.at[widx], srcv)
        pltpu.sync_copy(dst_hbm.at[widx], dstv)

        def zrow(r, carry):
            for j in range(D // LANES):
                rows[r, pl.ds(j * LANES, LANES)] = jnp.zeros((LANES,), _f32)
            return carry

        lax.fori_loop(0, CH, zrow, 0)
        for j in range(RPT_ACC // CH):
            pltpu.sync_copy(rows, acc.at[pl.ds(s * RPT_ACC + j * CH, CH)])
        plsc.subcore_barrier()

        def step(k, carry):
            pltpu.async_copy(hw_hbm.at[srcv.at[k]], rows, gsem).wait()
            pltpu.sync_copy(rows, acc.at[dstv.at[k]], add=True)
            return carry

        lax.fori_loop(0, K, step, 0)
        plsc.subcore_barrier()
        pltpu.sync_copy(acc.at[pl.ds(s * RPT_ACC, RPT_ACC)],
                        out_hbm.at[c, pl.ds(s * RPT_ACC, RPT_ACC)])

    return spmm_kernel


BM = 1000


def _dinv_tc(deg_stacked):
    def body(d_ref, o_ref):
        deg = d_ref[0:80, :] + d_ref[80:160, :]
        o_ref[...] = jnp.where(deg > 0.0, lax.rsqrt(deg), 0.0)

    return pl.pallas_call(
        body, out_shape=jax.ShapeDtypeStruct((80, 128), _f32))(deg_stacked)


def _mm_scale_pool(x, W, dinv_col, batch2d):
    def body(x_ref, w_ref, dv_ref, b_ref, hw_ref, pool_ref):
        i = pl.program_id(0)
        xb = x_ref[...]
        hw = jnp.dot(xb, w_ref[...], preferred_element_type=_f32)
        hw_ref[...] = hw * dv_ref[...]
        onehot = (b_ref[...] == lax.broadcasted_iota(_i32, (1, G), 1)
                  ).astype(_f32)
        p = lax.dot_general(onehot, xb, (((0,), (0,)), ((), ())),
                            preferred_element_type=_f32)

        @pl.when(i == 0)
        def _():
            pool_ref[...] = p

        @pl.when(i > 0)
        def _():
            pool_ref[...] += p

    return pl.pallas_call(
        body,
        grid=(N // BM,),
        in_specs=[
            pl.BlockSpec((BM, D), lambda i: (i, 0)),
            pl.BlockSpec((D, D), lambda i: (0, 0)),
            pl.BlockSpec((BM, 1), lambda i: (i, 0)),
            pl.BlockSpec((BM, 1), lambda i: (i, 0)),
        ],
        out_specs=[
            pl.BlockSpec((BM, D), lambda i: (i, 0)),
            pl.BlockSpec((G, D), lambda i: (0, 0)),
        ],
        out_shape=[
            jax.ShapeDtypeStruct((N, D), _f32),
            jax.ShapeDtypeStruct((G, D), _f32),
        ],
    )(x, W, dinv_col, batch2d)


def _t_sums(p0, p1, dinv_col, bias_row):
    def body(p0_ref, p1_ref, dv_ref, b_ref, t_ref, s_ref):
        i = pl.program_id(0)
        t = (p0_ref[...] + p1_ref[...]) * dv_ref[...] + b_ref[...]
        t_ref[...] = t
        st = jnp.sum(t, axis=0, keepdims=True)
        st2 = jnp.sum(t * t, axis=0, keepdims=True)
        blk = jnp.concatenate(
            [st, st2, jnp.zeros((6, 128), _f32)], axis=0)

        @pl.when(i == 0)
        def _():
            s_ref[...] = blk

        @pl.when(i > 0)
        def _():
            s_ref[...] += blk

    return pl.pallas_call(
        body,
        grid=(N // BM,),
        in_specs=[
            pl.BlockSpec((BM, D), lambda i: (i, 0)),
            pl.BlockSpec((BM, D), lambda i: (i, 0)),
            pl.BlockSpec((BM, 1), lambda i: (i, 0)),
            pl.BlockSpec((1, D), lambda i: (0, 0)),
        ],
        out_specs=[
            pl.BlockSpec((BM, D), lambda i: (i, 0)),
            pl.BlockSpec((8, D), lambda i: (0, 0)),
        ],
        out_shape=[
            jax.ShapeDtypeStruct((N, D), _f32),
            jax.ShapeDtypeStruct((8, D), _f32),
        ],
    )(p0, p1, dinv_col, bias_row)


def _bn_stats(s_ref):
    m = s_ref[0:1, :] * (1.0 / N)
    ex2 = s_ref[1:2, :] * (1.0 / N)
    var = ex2 - m * m
    inv = lax.rsqrt(var + 1e-5)
    return m, inv


def _bn_mm_pool(t, sums, gamma_row, beta_row, W, dinv_col, batch2d):
    def body(t_ref, s_ref, g_ref, be_ref, w_ref, dv_ref, b_ref,
             hw_ref, pool_ref):
        i = pl.program_id(0)
        m, inv = _bn_stats(s_ref)
        h = jnp.maximum((t_ref[...] - m) * inv * g_ref[...] + be_ref[...],
                        0.0)
        hw = jnp.dot(h, w_ref[...], preferred_element_type=_f32)
        hw_ref[...] = hw * dv_ref[...]
        onehot = (b_ref[...] == lax.broadcasted_iota(_i32, (1, G), 1)
                  ).astype(_f32)
        p = lax.dot_general(onehot, h, (((0,), (0,)), ((), ())),
                            preferred_element_type=_f32)

        @pl.when(i == 0)
        def _():
            pool_ref[...] = p

        @pl.when(i > 0)
        def _():
            pool_ref[...] += p

    return pl.pallas_call(
        body,
        grid=(N // BM,),
        in_specs=[
            pl.BlockSpec((BM, D), lambda i: (i, 0)),
            pl.BlockSpec((8, D), lambda i: (0, 0)),
            pl.BlockSpec((1, D), lambda i: (0, 0)),
            pl.BlockSpec((1, D), lambda i: (0, 0)),
            pl.BlockSpec((D, D), lambda i: (0, 0)),
            pl.BlockSpec((BM, 1), lambda i: (i, 0)),
            pl.BlockSpec((BM, 1), lambda i: (i, 0)),
        ],
        out_specs=[
            pl.BlockSpec((BM, D), lambda i: (i, 0)),
            pl.BlockSpec((G, D), lambda i: (0, 0)),
        ],
        out_shape=[
            jax.ShapeDtypeStruct((N, D), _f32),
            jax.ShapeDtypeStruct((G, D), _f32),
        ],
    )(t, sums, gamma_row, beta_row, W, dinv_col, batch2d)


def _bn_pool(t, sums, gamma_row, beta_row, batch2d):
    def body(t_ref, s_ref, g_ref, be_ref, b_ref, pool_ref):
        i = pl.program_id(0)
        m, inv = _bn_stats(s_ref)
        h = jnp.maximum((t_ref[...] - m) * inv * g_ref[...] + be_ref[...],
                        0.0)
        onehot = (b_ref[...] == lax.broadcasted_iota(_i32, (1, G), 1)
                  ).astype(_f32)
        p = lax.dot_general(onehot, h, (((0,), (0,)), ((), ())),
                            preferred_element_type=_f32)

        @pl.when(i == 0)
        def _():
            pool_ref[...] = p

        @pl.when(i > 0)
        def _():
            pool_ref[...] += p

    return pl.pallas_call(
        body,
        grid=(N // BM,),
        in_specs=[
            pl.BlockSpec((BM, D), lambda i: (i, 0)),
            pl.BlockSpec((8, D), lambda i: (0, 0)),
            pl.BlockSpec((1, D), lambda i: (0, 0)),
            pl.BlockSpec((1, D), lambda i: (0, 0)),
            pl.BlockSpec((BM, 1), lambda i: (i, 0)),
        ],
        out_specs=pl.BlockSpec((G, D), lambda i: (0, 0)),
        out_shape=jax.ShapeDtypeStruct((G, D), _f32),
    )(t, sums, gamma_row, beta_row, batch2d)


def _final(px, p1, p2, Wp0, Wp1, Wp2, b0r, b1r, b2r):
    def body(px_ref, p1_ref, p2_ref, w0_ref, w1_ref, w2_ref,
             b0_ref, b1_ref, b2_ref, o_ref):
        o_ref[...] = (
            jnp.dot(px_ref[...], w0_ref[...], preferred_element_type=_f32)
            + jnp.dot(p1_ref[...], w1_ref[...], preferred_element_type=_f32)
            + jnp.dot(p2_ref[...], w2_ref[...], preferred_element_type=_f32)
            + b0_ref[...] + b1_ref[...] + b2_ref[...])

    return pl.pallas_call(
        body, out_shape=jax.ShapeDtypeStruct((G, D), _f32),
    )(px, p1, p2, Wp0, Wp1, Wp2, b0r, b1r, b2r)


def kernel(x, edge_index, batch, W1, b1, W2, b2, gamma1, beta1,
           gamma2, beta2, Wp0, bp0, Wp1, bp1, Wp2, bp2):
    E = edge_index.shape[1]
    etot = E + N
    K = -(-etot // (NW * CH))
    if K % 2:
        K += 1
    ep = NW * K * CH

    loop = jnp.arange(N, dtype=_i32)
    src = jnp.concatenate(
        [edge_index[0].astype(_i32), loop,
         jnp.zeros((ep - etot,), _i32)]).reshape(NW, K, CH)
    dst = jnp.concatenate(
        [edge_index[1].astype(_i32), loop,
         jnp.full((ep - etot,), N, _i32)]).reshape(NW, K, CH)

    degp = _deg_sc(K)(dst)
    dinv_tiles = _dinv_tc(degp.reshape(2 * 80, 128))
    dinv_col = dinv_tiles.reshape(NPAD)[:N].reshape(N, 1)
    batch2d = batch.astype(_i32).reshape(N, 1)

    hw1, poolx = _mm_scale_pool(x, W1, dinv_col, batch2d)
    s1 = _spmm_sc(K)(hw1, src, dst)
    t1, sums1 = _t_sums(s1[0, :N], s1[1, :N], dinv_col, b1.reshape(1, D))
    hw2, pool1 = _bn_mm_pool(t1, sums1, gamma1.reshape(1, D),
                             beta1.reshape(1, D), W2, dinv_col, batch2d)
    s2 = _spmm_sc(K)(hw2, src, dst)
    t2, sums2 = _t_sums(s2[0, :N], s2[1, :N], dinv_col, b2.reshape(1, D))
    pool2 = _bn_pool(t2, sums2, gamma2.reshape(1, D),
                     beta2.reshape(1, D), batch2d)
    return _final(poolx, pool1, pool2, Wp0, Wp1, Wp2,
                  bp0.reshape(1, D), bp1.reshape(1, D), bp2.reshape(1, D))
